# Initial kernel scaffold; baseline (speedup 1.0000x reference)
import functools

import jax
import jax.numpy as jnp
from jax import lax
from jax.experimental import pallas as pl
from jax.experimental.pallas import tpu as pltpu
from jax.experimental.pallas import tpu_sc as plsc

N_NODES = 10000
D = 128

NC = 2
NS = 16
CHUNK = 128
CHUNKS_PER_TILE = 80
EDGES_PAD = NC * NS * CHUNKS_PER_TILE * CHUNK
ACC_ROWS = 10240
ROWS_PER_TILE = ACC_ROWS // NS
DUMMY = N_NODES
CNT_W = 16

_mesh = plsc.VectorSubcoreMesh(core_axis_name="c", subcore_axis_name="s")


def _sc_body(with_cnt, y_hbm, src_hbm, dst_hbm, *refs):
    if with_cnt:
        (p_hbm, cnt_hbm, srcv, dstv, rows0, rows1, acc, sem0, sem1, zbuf,
         onesv, cacc, zcnt) = refs
    else:
        p_hbm, srcv, dstv, rows0, rows1, acc, sem0, sem1, zbuf = refs

    c = lax.axis_index("c")
    s = lax.axis_index("s")
    wid = s * NC + c
    base = s * ROWS_PER_TILE

    zero16 = jnp.zeros((16,), jnp.float32)
    for i in range(zbuf.shape[0]):
        for j in range(D // 16):
            zbuf[i, pl.ds(j * 16, 16)] = zero16
    if with_cnt:
        one16 = jnp.ones((16,), jnp.float32)
        for i in range(CHUNK):
            onesv[i, :] = one16
        for i in range(zcnt.shape[0]):
            zcnt[i, :] = zero16

    pltpu.sync_copy(src_hbm.at[pl.ds(wid * CHUNKS_PER_TILE, CHUNKS_PER_TILE)], srcv)
    pltpu.sync_copy(dst_hbm.at[pl.ds(wid * CHUNKS_PER_TILE, CHUNKS_PER_TILE)], dstv)

    for t in range(ROWS_PER_TILE // zbuf.shape[0]):
        pltpu.sync_copy(zbuf, acc.at[pl.ds(base + t * zbuf.shape[0], zbuf.shape[0])])
    if with_cnt:
        for t in range(ROWS_PER_TILE // zcnt.shape[0]):
            pltpu.sync_copy(zcnt, cacc.at[pl.ds(base + t * zcnt.shape[0], zcnt.shape[0])])
    plsc.subcore_barrier()

    pltpu.async_copy(y_hbm.at[srcv.at[0]], rows0, sem0)
    pltpu.async_copy(y_hbm.at[srcv.at[1]], rows1, sem1)

    def step(i, carry):
        for b, (rows, sem) in enumerate(((rows0, sem0), (rows1, sem1))):
            j = i * 2 + b
            pltpu.make_async_copy(y_hbm.at[srcv.at[j]], rows, sem).wait()
            pltpu.sync_copy(rows, acc.at[dstv.at[j]], add=True)
            if with_cnt:
                pltpu.sync_copy(onesv, cacc.at[dstv.at[j]], add=True)

            @pl.when(j + 2 < CHUNKS_PER_TILE)
            def _():
                pltpu.async_copy(y_hbm.at[srcv.at[j + 2]], rows, sem)
        return carry

    lax.fori_loop(0, CHUNKS_PER_TILE // 2, step, 0)
    plsc.subcore_barrier()

    pltpu.sync_copy(acc.at[pl.ds(base, ROWS_PER_TILE)],
                    p_hbm.at[c, pl.ds(base, ROWS_PER_TILE)])
    if with_cnt:
        pltpu.sync_copy(cacc.at[pl.ds(base, ROWS_PER_TILE)],
                        cnt_hbm.at[c, pl.ds(base, ROWS_PER_TILE)])


def _make_sc_kernel(with_cnt):
    out_type = [jax.ShapeDtypeStruct((NC, ACC_ROWS, D), jnp.float32)]
    scratch = [
        pltpu.VMEM((CHUNKS_PER_TILE, CHUNK), jnp.int32),
        pltpu.VMEM((CHUNKS_PER_TILE, CHUNK), jnp.int32),
        pltpu.VMEM((CHUNK, D), jnp.float32),
        pltpu.VMEM((CHUNK, D), jnp.float32),
        pltpu.VMEM_SHARED((ACC_ROWS, D), jnp.float32),
        pltpu.SemaphoreType.DMA,
        pltpu.SemaphoreType.DMA,
        pltpu.VMEM((16, D), jnp.float32),
    ]
    if with_cnt:
        out_type = out_type + [jax.ShapeDtypeStruct((NC, ACC_ROWS, CNT_W), jnp.float32)]
        scratch = scratch + [
            pltpu.VMEM((CHUNK, CNT_W), jnp.float32),
            pltpu.VMEM_SHARED((ACC_ROWS, CNT_W), jnp.float32),
            pltpu.VMEM((64, CNT_W), jnp.float32),
        ]
    return pl.kernel(
        functools.partial(_sc_body, with_cnt),
        out_type=out_type,
        mesh=_mesh,
        scratch_types=scratch,
    )


_sc_scatter_cnt = _make_sc_kernel(True)
_sc_scatter = _make_sc_kernel(False)


_BR = 1000
_GRID = (N_NODES // _BR,)


def _row_spec(w=D):
    return pl.BlockSpec((_BR, w), lambda i: (i, 0))


def _full_spec(shape):
    return pl.BlockSpec(shape, lambda i: (0,) * len(shape))


def _tc1_body(x_ref, wl_ref, wr_ref, b_ref, y_ref, z_ref):
    x = x_ref[...]
    y_ref[...] = jnp.dot(x, wl_ref[...], preferred_element_type=jnp.float32)
    z_ref[...] = jnp.dot(x, wr_ref[...], preferred_element_type=jnp.float32) + b_ref[...]


def _tc2_body(pa_ref, pb_ref, ca_ref, cb_ref, z1_ref, wl_ref, wr_ref, b_ref,
              y_ref, z_ref):
    cnt = ca_ref[...][:, 0:1] + cb_ref[...][:, 0:1]
    mean = (pa_ref[...] + pb_ref[...]) / jnp.clip(cnt, 1.0, None)
    h = jnp.maximum(mean + z1_ref[...], 0.0)
    y_ref[...] = jnp.dot(h, wl_ref[...], preferred_element_type=jnp.float32)
    z_ref[...] = jnp.dot(h, wr_ref[...], preferred_element_type=jnp.float32) + b_ref[...]


def _tc3_body(pa_ref, pb_ref, ca_ref, cb_ref, z2_ref, o_ref):
    cnt = ca_ref[...][:, 0:1] + cb_ref[...][:, 0:1]
    mean = (pa_ref[...] + pb_ref[...]) / jnp.clip(cnt, 1.0, None)
    o_ref[...] = mean + z2_ref[...]


_tc1 = pl.pallas_call(
    _tc1_body,
    grid=_GRID,
    in_specs=[_row_spec(), _full_spec((D, D)), _full_spec((D, D)),
              _full_spec((1, D))],
    out_specs=[_row_spec(), _row_spec()],
    out_shape=[jax.ShapeDtypeStruct((N_NODES, D), jnp.float32)] * 2,
)

_tc2 = pl.pallas_call(
    _tc2_body,
    grid=_GRID,
    in_specs=[_row_spec(), _row_spec(), _row_spec(CNT_W), _row_spec(CNT_W),
              _row_spec(), _full_spec((D, D)), _full_spec((D, D)),
              _full_spec((1, D))],
    out_specs=[_row_spec(), _row_spec()],
    out_shape=[jax.ShapeDtypeStruct((N_NODES, D), jnp.float32)] * 2,
)

_tc3 = pl.pallas_call(
    _tc3_body,
    grid=_GRID,
    in_specs=[_row_spec(), _row_spec(), _row_spec(CNT_W), _row_spec(CNT_W),
              _row_spec()],
    out_specs=_row_spec(),
    out_shape=jax.ShapeDtypeStruct((N_NODES, D), jnp.float32),
)


@jax.jit
def kernel(x, edge_index, W1_l, b1, W1_r, W2_l, b2, W2_r):
    n_edges = edge_index.shape[1]
    pad = EDGES_PAD - n_edges
    src = jnp.concatenate(
        [edge_index[0].astype(jnp.int32), jnp.zeros((pad,), jnp.int32)]
    ).reshape(-1, CHUNK)
    dst = jnp.concatenate(
        [edge_index[1].astype(jnp.int32), jnp.full((pad,), DUMMY, jnp.int32)]
    ).reshape(-1, CHUNK)

    y1, z1 = _tc1(x, W1_l.T, W1_r.T, b1[None, :])
    p1, cnt = _sc_scatter_cnt(y1, src, dst)
    y2, z2 = _tc2(p1[0], p1[1], cnt[0], cnt[1], z1, W2_l.T, W2_r.T, b2[None, :])
    (p2,) = _sc_scatter(y2, src, dst)
    out = _tc3(p2[0], p2[1], cnt[0], cnt[1], z2)
    return out


# speedup vs baseline: 4.9310x; 4.9310x over previous
import jax
import jax.numpy as jnp
from jax import lax
from jax.experimental import pallas as pl
from jax.experimental.pallas import tpu as pltpu
from jax.experimental.pallas import tpu_sc as plsc

N_NODES = 10000
D = 128
DH = D // 2

NC = 2
NS = 16
CHUNK = 128
EDGES_PAD = 327680
CHUNKS_TOTAL = EDGES_PAD // CHUNK
SCAT_CHUNKS = CHUNKS_TOTAL // NS
CNT_CHUNKS = CHUNKS_TOTAL // (NC * NS)
ACC_ROWS = 10240
ROWS_PER_TILE = ACC_ROWS // NS
DUMMY = N_NODES
CNT_W = 16

_mesh = plsc.VectorSubcoreMesh(core_axis_name="c", subcore_axis_name="s")


def _sc_scatter_body(ya_hbm, yb_hbm, src_hbm, dst_hbm, p_hbm,
                     srcv, dstv, rows0, rows1, acc, sem0, sem1, zbuf):
    c = lax.axis_index("c")
    s = lax.axis_index("s")
    base = s * ROWS_PER_TILE

    zero16 = jnp.zeros((16,), jnp.float32)
    for i in range(zbuf.shape[0]):
        for j in range(DH // 16):
            zbuf[i, pl.ds(j * 16, 16)] = zero16

    pltpu.sync_copy(src_hbm.at[pl.ds(s * SCAT_CHUNKS, SCAT_CHUNKS)], srcv)
    pltpu.sync_copy(dst_hbm.at[pl.ds(s * SCAT_CHUNKS, SCAT_CHUNKS)], dstv)

    for t in range(ROWS_PER_TILE // zbuf.shape[0]):
        pltpu.sync_copy(zbuf, acc.at[pl.ds(base + t * zbuf.shape[0], zbuf.shape[0])])
    plsc.subcore_barrier()

    def run(y_hbm):
        pltpu.async_copy(y_hbm.at[srcv.at[0]], rows0, sem0)
        pltpu.async_copy(y_hbm.at[srcv.at[1]], rows1, sem1)

        def step(i, carry):
            for b, (rows, sem) in enumerate(((rows0, sem0), (rows1, sem1))):
                j = i * 2 + b
                pltpu.make_async_copy(y_hbm.at[srcv.at[j]], rows, sem).wait()
                pltpu.sync_copy(rows, acc.at[dstv.at[j]], add=True)

                @pl.when(j + 2 < SCAT_CHUNKS)
                def _():
                    pltpu.async_copy(y_hbm.at[srcv.at[j + 2]], rows, sem)
            return carry

        lax.fori_loop(0, SCAT_CHUNKS // 2, step, 0)

    @pl.when(c == 0)
    def _():
        run(ya_hbm)

    @pl.when(c == 1)
    def _():
        run(yb_hbm)

    plsc.subcore_barrier()
    pltpu.sync_copy(acc.at[pl.ds(base, ROWS_PER_TILE)],
                    p_hbm.at[c, pl.ds(base, ROWS_PER_TILE)])


_sc_scatter = pl.kernel(
    _sc_scatter_body,
    out_type=[jax.ShapeDtypeStruct((NC, ACC_ROWS, DH), jnp.float32)],
    mesh=_mesh,
    scratch_types=[
        pltpu.VMEM((SCAT_CHUNKS, CHUNK), jnp.int32),
        pltpu.VMEM((SCAT_CHUNKS, CHUNK), jnp.int32),
        pltpu.VMEM((CHUNK, DH), jnp.float32),
        pltpu.VMEM((CHUNK, DH), jnp.float32),
        pltpu.VMEM_SHARED((ACC_ROWS, DH), jnp.float32),
        pltpu.SemaphoreType.DMA,
        pltpu.SemaphoreType.DMA,
        pltpu.VMEM((16, DH), jnp.float32),
    ],
    compiler_params=pltpu.CompilerParams(use_tc_tiling_on_sc=False),
)


def _sc_counts_body(dst_hbm, cnt_hbm, dstv, onesv, cacc, zcnt):
    c = lax.axis_index("c")
    s = lax.axis_index("s")
    wid = s * NC + c
    base = s * ROWS_PER_TILE

    zero16 = jnp.zeros((16,), jnp.float32)
    one16 = jnp.ones((16,), jnp.float32)
    for i in range(CHUNK):
        onesv[i, :] = one16
    for i in range(zcnt.shape[0]):
        zcnt[i, :] = zero16

    pltpu.sync_copy(dst_hbm.at[pl.ds(wid * CNT_CHUNKS, CNT_CHUNKS)], dstv)
    for t in range(ROWS_PER_TILE // zcnt.shape[0]):
        pltpu.sync_copy(zcnt, cacc.at[pl.ds(base + t * zcnt.shape[0], zcnt.shape[0])])
    plsc.subcore_barrier()

    def step(j, carry):
        pltpu.sync_copy(onesv, cacc.at[dstv.at[j]], add=True)
        return carry

    lax.fori_loop(0, CNT_CHUNKS, step, 0)
    plsc.subcore_barrier()

    pltpu.sync_copy(cacc.at[pl.ds(base, ROWS_PER_TILE)],
                    cnt_hbm.at[c, pl.ds(base, ROWS_PER_TILE)])


_sc_counts = pl.kernel(
    _sc_counts_body,
    out_type=[jax.ShapeDtypeStruct((NC, ACC_ROWS, CNT_W), jnp.float32)],
    mesh=_mesh,
    scratch_types=[
        pltpu.VMEM((CNT_CHUNKS, CHUNK), jnp.int32),
        pltpu.VMEM((CHUNK, CNT_W), jnp.float32),
        pltpu.VMEM_SHARED((ACC_ROWS, CNT_W), jnp.float32),
        pltpu.VMEM((64, CNT_W), jnp.float32),
    ],
    compiler_params=pltpu.CompilerParams(use_tc_tiling_on_sc=False),
)


_BR = 1000
_GRID = (N_NODES // _BR,)


def _row_spec(w=D):
    return pl.BlockSpec((_BR, w), lambda i: (i, 0))


def _full_spec(shape):
    return pl.BlockSpec(shape, lambda i: (0,) * len(shape))


def _tc1_body(x_ref, wl_ref, wr_ref, b_ref, ya_ref, yb_ref, z_ref):
    x = x_ref[...]
    y = jnp.dot(x, wl_ref[...], preferred_element_type=jnp.float32)
    ya_ref[...] = y[:, :DH]
    yb_ref[...] = y[:, DH:]
    z_ref[...] = jnp.dot(x, wr_ref[...], preferred_element_type=jnp.float32) + b_ref[...]


def _tc2_body(pa_ref, pb_ref, ca_ref, cb_ref, z1_ref, wl_ref, wr_ref, b_ref,
              ya_ref, yb_ref, z_ref):
    inv = 1.0 / jnp.clip(ca_ref[...][:, 0:1] + cb_ref[...][:, 0:1], 1.0, None)
    mean = jnp.concatenate([pa_ref[...], pb_ref[...]], axis=1) * inv
    h = jnp.maximum(mean + z1_ref[...], 0.0)
    y = jnp.dot(h, wl_ref[...], preferred_element_type=jnp.float32)
    ya_ref[...] = y[:, :DH]
    yb_ref[...] = y[:, DH:]
    z_ref[...] = jnp.dot(h, wr_ref[...], preferred_element_type=jnp.float32) + b_ref[...]


def _tc3_body(pa_ref, pb_ref, ca_ref, cb_ref, z2_ref, o_ref):
    inv = 1.0 / jnp.clip(ca_ref[...][:, 0:1] + cb_ref[...][:, 0:1], 1.0, None)
    mean = jnp.concatenate([pa_ref[...], pb_ref[...]], axis=1) * inv
    o_ref[...] = mean + z2_ref[...]


_tc1 = pl.pallas_call(
    _tc1_body,
    grid=_GRID,
    in_specs=[_row_spec(), _full_spec((D, D)), _full_spec((D, D)),
              _full_spec((1, D))],
    out_specs=[_row_spec(DH), _row_spec(DH), _row_spec()],
    out_shape=[jax.ShapeDtypeStruct((N_NODES, DH), jnp.float32),
               jax.ShapeDtypeStruct((N_NODES, DH), jnp.float32),
               jax.ShapeDtypeStruct((N_NODES, D), jnp.float32)],
)

_tc2 = pl.pallas_call(
    _tc2_body,
    grid=_GRID,
    in_specs=[_row_spec(DH), _row_spec(DH), _row_spec(CNT_W), _row_spec(CNT_W),
              _row_spec(), _full_spec((D, D)), _full_spec((D, D)),
              _full_spec((1, D))],
    out_specs=[_row_spec(DH), _row_spec(DH), _row_spec()],
    out_shape=[jax.ShapeDtypeStruct((N_NODES, DH), jnp.float32),
               jax.ShapeDtypeStruct((N_NODES, DH), jnp.float32),
               jax.ShapeDtypeStruct((N_NODES, D), jnp.float32)],
)

_tc3 = pl.pallas_call(
    _tc3_body,
    grid=_GRID,
    in_specs=[_row_spec(DH), _row_spec(DH), _row_spec(CNT_W), _row_spec(CNT_W),
              _row_spec()],
    out_specs=_row_spec(),
    out_shape=jax.ShapeDtypeStruct((N_NODES, D), jnp.float32),
)


@jax.jit
def kernel(x, edge_index, W1_l, b1, W1_r, W2_l, b2, W2_r):
    n_edges = edge_index.shape[1]
    pad = EDGES_PAD - n_edges
    src = jnp.concatenate(
        [edge_index[0].astype(jnp.int32), jnp.zeros((pad,), jnp.int32)]
    ).reshape(-1, CHUNK)
    dst = jnp.concatenate(
        [edge_index[1].astype(jnp.int32), jnp.full((pad,), DUMMY, jnp.int32)]
    ).reshape(-1, CHUNK)

    (cnt,) = _sc_counts(dst)
    ya1, yb1, z1 = _tc1(x, W1_l.T, W1_r.T, b1[None, :])
    (p1,) = _sc_scatter(ya1, yb1, src, dst)
    ya2, yb2, z2 = _tc2(p1[0], p1[1], cnt[0], cnt[1], z1,
                        W2_l.T, W2_r.T, b2[None, :])
    (p2,) = _sc_scatter(ya2, yb2, src, dst)
    out = _tc3(p2[0], p2[1], cnt[0], cnt[1], z2)
    return out
